# baseline (speedup 1.0000x reference)
import functools

import jax
import jax.numpy as jnp
from jax import lax
from jax.experimental import pallas as pl
from jax.experimental.pallas import tpu as pltpu
from jax.experimental.pallas import tpu_sc as plsc

NC = 2
NS = 16
NW = NC * NS

K = 80
NBUF = 3
NIDX = 6
NCH0 = 126
NCH1 = 126


def _sc_scatter_add(x, edges):
    _, n_nodes, d_feat = x.shape
    n_pad = ((n_nodes + NS * 8 - 1) // (NS * 8)) * (NS * 8)
    rows_per_tile = n_pad // NS

    mesh = plsc.VectorSubcoreMesh(
        core_axis_name="c", subcore_axis_name="s", num_cores=NC, num_subcores=NS
    )

    @functools.partial(
        pl.kernel,
        out_type=jax.ShapeDtypeStruct((NC, n_pad, d_feat), jnp.float32),
        mesh=mesh,
        scratch_types=[
            pltpu.VMEM((NIDX, 2, K), jnp.int32),
            pltpu.VMEM((NBUF, K, d_feat), jnp.float32),
            pltpu.VMEM_SHARED((n_pad, d_feat), jnp.float32),
            [pltpu.SemaphoreType.DMA] * NIDX,
            [pltpu.SemaphoreType.DMA] * NBUF,
        ],
    )
    def sc_kernel(x_hbm, e_hbm, out_hbm, idx_v, rows_v, acc, isems, gsems):
        c = lax.axis_index("c")
        s = lax.axis_index("s")
        x_mine = x_hbm.at[c]
        n_chunks = jnp.where(c == 0, NCH0, NCH1)
        chunk_base = jnp.where(c == 0, s * NCH0, NS * NCH0 + s * NCH1)

        scope_zero = jax.named_scope("ph_zero"); scope_zero.__enter__()
        def zero_row(i, carry):
            for j in range(d_feat // 16):
                rows_v[0, i, pl.ds(j * 16, 16)] = jnp.zeros((16,), jnp.float32)
            return carry

        lax.fori_loop(0, K, zero_row, 0)
        row_base = s * rows_per_tile
        nfull = rows_per_tile // K
        for b in range(nfull):
            pltpu.sync_copy(rows_v.at[0], acc.at[pl.ds(row_base + b * K, K)])
        rem = rows_per_tile - nfull * K
        if rem:
            pltpu.sync_copy(
                rows_v.at[0, pl.ds(0, rem)],
                acc.at[pl.ds(row_base + nfull * K, rem)],
            )
        plsc.subcore_barrier()
        scope_zero.__exit__(None, None, None)
        scope_main = jax.named_scope("ph_main"); scope_main.__enter__()

        def idx_start(g, i):
            pltpu.async_copy(e_hbm.at[chunk_base + g], idx_v.at[i], isems[i])

        def idx_wait(i):
            pltpu.make_async_copy(e_hbm.at[0], idx_v.at[i], isems[i]).wait()

        def gather_start(g, i, b):
            pltpu.async_copy(x_mine.at[idx_v.at[i, 0]], rows_v.at[b], gsems[b])

        def gather_wait(b):
            pltpu.make_async_copy(
                x_mine.at[idx_v.at[0, 0]], rows_v.at[b], gsems[b]
            ).wait()

        def scatter(i, b):
            pltpu.sync_copy(rows_v.at[b], acc.at[idx_v.at[i, 1]], add=True)

        for g in range(NIDX):
            idx_start(g, g)
        for g in range(NBUF):
            idx_wait(g)
            gather_start(g, g, g)

        @pl.loop(0, n_chunks - NIDX, step=NIDX)
        def _(g0):
            for b in range(NIDX):
                g = g0 + b
                gather_wait(b % NBUF)
                scatter(b, b % NBUF)
                idx_start(g + NIDX, b)
                idx_wait((b + NBUF) % NIDX)
                gather_start(g + NBUF, (b + NBUF) % NIDX, b % NBUF)

        for b in range(NIDX):
            g = n_chunks - NIDX + b
            gather_wait(b % NBUF)
            scatter(b, b % NBUF)
            if b + NBUF < NIDX:
                idx_wait((b + NBUF) % NIDX)
                gather_start(g + NBUF, (b + NBUF) % NIDX, b % NBUF)
        plsc.subcore_barrier()
        scope_main.__exit__(None, None, None)

        with jax.named_scope("ph_writeback"):
            pltpu.sync_copy(
                acc.at[pl.ds(row_base, rows_per_tile)],
                out_hbm.at[c, pl.ds(row_base, rows_per_tile)],
            )

    return sc_kernel(x, edges)


def _finalize(x, acc):
    n_nodes, d_feat = x.shape
    br = 1000

    def body(x_ref, acc_ref, o_ref):
        o_ref[:, :d_feat] = x_ref[...]
        o_ref[:, d_feat:] = acc_ref[0] + acc_ref[1]

    return pl.pallas_call(
        body,
        grid=(n_nodes // br,),
        in_specs=[
            pl.BlockSpec((br, d_feat), lambda i: (i, 0)),
            pl.BlockSpec((NC, br, d_feat), lambda i: (0, i, 0)),
        ],
        out_specs=pl.BlockSpec((br, 2 * d_feat), lambda i: (i, 0)),
        out_shape=jax.ShapeDtypeStruct((n_nodes, 2 * d_feat), jnp.float32),
    )(x, acc)


@jax.jit
def _run(x, edges):
    xx = jnp.broadcast_to(x, (NC,) + x.shape)
    acc = _sc_scatter_add(xx, edges)
    return _finalize(x, acc)


def kernel(x, edge_index, batch_size):
    n_nodes = x.shape[0]
    n_edges = edge_index.shape[1]
    ei = edge_index.astype(jnp.int32)
    n_pad_rows = ((n_nodes + NS * 8 - 1) // (NS * 8)) * (NS * 8) - n_nodes
    n_padded = NS * (NCH0 + NCH1) * K
    pad = n_padded - n_edges
    src = jnp.concatenate([ei[0], jnp.zeros((pad,), jnp.int32)])
    dst = jnp.concatenate(
        [ei[1], n_nodes + (jnp.arange(pad, dtype=jnp.int32) % n_pad_rows)]
    )
    edges = jnp.stack(
        [src.reshape(-1, K), dst.reshape(-1, K)], axis=1
    )
    return _run(x, edges)


# speedup vs baseline: 1.0111x; 1.0111x over previous
import functools

import jax
import jax.numpy as jnp
from jax import lax
from jax.experimental import pallas as pl
from jax.experimental.pallas import tpu as pltpu
from jax.experimental.pallas import tpu_sc as plsc

NC = 2
NS = 16
NW = NC * NS

K = 48
NBUF = 5
NIDX = 10
NCH0 = 210
NCH1 = 210


def _sc_scatter_add(x, edges):
    _, n_nodes, d_feat = x.shape
    n_pad = ((n_nodes + NS * 8 - 1) // (NS * 8)) * (NS * 8)
    rows_per_tile = n_pad // NS

    mesh = plsc.VectorSubcoreMesh(
        core_axis_name="c", subcore_axis_name="s", num_cores=NC, num_subcores=NS
    )

    @functools.partial(
        pl.kernel,
        out_type=jax.ShapeDtypeStruct((NC, n_pad, d_feat), jnp.float32),
        mesh=mesh,
        scratch_types=[
            pltpu.VMEM((NIDX, 2, K), jnp.int32),
            pltpu.VMEM((NBUF, K, d_feat), jnp.float32),
            pltpu.VMEM_SHARED((n_pad, d_feat), jnp.float32),
            [pltpu.SemaphoreType.DMA] * NIDX,
            [pltpu.SemaphoreType.DMA] * NBUF,
        ],
    )
    def sc_kernel(x_hbm, e_hbm, out_hbm, idx_v, rows_v, acc, isems, gsems):
        c = lax.axis_index("c")
        s = lax.axis_index("s")
        x_mine = x_hbm.at[c]
        n_chunks = jnp.where(c == 0, NCH0, NCH1)
        chunk_base = jnp.where(c == 0, s * NCH0, NS * NCH0 + s * NCH1)

        scope_zero = jax.named_scope("ph_zero"); scope_zero.__enter__()
        def zero_row(i, carry):
            for j in range(d_feat // 16):
                rows_v[0, i, pl.ds(j * 16, 16)] = jnp.zeros((16,), jnp.float32)
            return carry

        lax.fori_loop(0, K, zero_row, 0)
        row_base = s * rows_per_tile
        nfull = rows_per_tile // K
        for b in range(nfull):
            pltpu.sync_copy(rows_v.at[0], acc.at[pl.ds(row_base + b * K, K)])
        rem = rows_per_tile - nfull * K
        if rem:
            pltpu.sync_copy(
                rows_v.at[0, pl.ds(0, rem)],
                acc.at[pl.ds(row_base + nfull * K, rem)],
            )
        plsc.subcore_barrier()
        scope_zero.__exit__(None, None, None)
        scope_main = jax.named_scope("ph_main"); scope_main.__enter__()

        def idx_start(g, i):
            pltpu.async_copy(e_hbm.at[chunk_base + g], idx_v.at[i], isems[i])

        def idx_wait(i):
            pltpu.make_async_copy(e_hbm.at[0], idx_v.at[i], isems[i]).wait()

        def gather_start(g, i, b):
            pltpu.async_copy(x_mine.at[idx_v.at[i, 0]], rows_v.at[b], gsems[b])

        def gather_wait(b):
            pltpu.make_async_copy(
                x_mine.at[idx_v.at[0, 0]], rows_v.at[b], gsems[b]
            ).wait()

        def scatter(i, b):
            pltpu.sync_copy(rows_v.at[b], acc.at[idx_v.at[i, 1]], add=True)

        for g in range(NIDX):
            idx_start(g, g)
        for g in range(NBUF):
            idx_wait(g)
            gather_start(g, g, g)

        @pl.loop(0, n_chunks - NIDX, step=NIDX)
        def _(g0):
            for b in range(NIDX):
                g = g0 + b
                gather_wait(b % NBUF)
                scatter(b, b % NBUF)
                idx_start(g + NIDX, b)
                idx_wait((b + NBUF) % NIDX)
                gather_start(g + NBUF, (b + NBUF) % NIDX, b % NBUF)

        for b in range(NIDX):
            g = n_chunks - NIDX + b
            gather_wait(b % NBUF)
            scatter(b, b % NBUF)
            if b + NBUF < NIDX:
                idx_wait((b + NBUF) % NIDX)
                gather_start(g + NBUF, (b + NBUF) % NIDX, b % NBUF)
        plsc.subcore_barrier()
        scope_main.__exit__(None, None, None)

        with jax.named_scope("ph_writeback"):
            pltpu.sync_copy(
                acc.at[pl.ds(row_base, rows_per_tile)],
                out_hbm.at[c, pl.ds(row_base, rows_per_tile)],
            )

    return sc_kernel(x, edges)


def _finalize(x, acc):
    n_nodes, d_feat = x.shape
    br = 1000

    def body(x_ref, acc_ref, o_ref):
        o_ref[:, :d_feat] = x_ref[...]
        o_ref[:, d_feat:] = acc_ref[0] + acc_ref[1]

    return pl.pallas_call(
        body,
        grid=(n_nodes // br,),
        in_specs=[
            pl.BlockSpec((br, d_feat), lambda i: (i, 0)),
            pl.BlockSpec((NC, br, d_feat), lambda i: (0, i, 0)),
        ],
        out_specs=pl.BlockSpec((br, 2 * d_feat), lambda i: (i, 0)),
        out_shape=jax.ShapeDtypeStruct((n_nodes, 2 * d_feat), jnp.float32),
    )(x, acc)


@jax.jit
def _run(x, edges):
    xx = jnp.broadcast_to(x, (NC,) + x.shape)
    acc = _sc_scatter_add(xx, edges)
    return _finalize(x, acc)


def kernel(x, edge_index, batch_size):
    n_nodes = x.shape[0]
    n_edges = edge_index.shape[1]
    ei = edge_index.astype(jnp.int32)
    n_pad_rows = ((n_nodes + NS * 8 - 1) // (NS * 8)) * (NS * 8) - n_nodes
    n_padded = NS * (NCH0 + NCH1) * K
    pad = n_padded - n_edges
    src = jnp.concatenate([ei[0], jnp.zeros((pad,), jnp.int32)])
    dst = jnp.concatenate(
        [ei[1], n_nodes + (jnp.arange(pad, dtype=jnp.int32) % n_pad_rows)]
    )
    edges = jnp.stack(
        [src.reshape(-1, K), dst.reshape(-1, K)], axis=1
    )
    return _run(x, edges)


# speedup vs baseline: 1.1587x; 1.1460x over previous
import functools

import jax
import jax.numpy as jnp
from jax import lax
from jax.experimental import pallas as pl
from jax.experimental.pallas import tpu as pltpu
from jax.experimental.pallas import tpu_sc as plsc

NC = 2
NS = 16
NW = NC * NS

K = 48
NBUF = 5
NIDX = 10
NCH0 = 390
NCH1 = 30


def _sc_scatter_add(x, edges):
    _, n_nodes, d_feat = x.shape
    n_pad = ((n_nodes + NS * 8 - 1) // (NS * 8)) * (NS * 8)
    rows_per_tile = n_pad // NS

    mesh = plsc.VectorSubcoreMesh(
        core_axis_name="c", subcore_axis_name="s", num_cores=NC, num_subcores=NS
    )

    @functools.partial(
        pl.kernel,
        out_type=jax.ShapeDtypeStruct((NC, n_pad, d_feat), jnp.float32),
        mesh=mesh,
        scratch_types=[
            pltpu.VMEM((NIDX, 2, K), jnp.int32),
            pltpu.VMEM((NBUF, K, d_feat), jnp.float32),
            pltpu.VMEM_SHARED((n_pad, d_feat), jnp.float32),
            [pltpu.SemaphoreType.DMA] * NIDX,
            [pltpu.SemaphoreType.DMA] * NBUF,
        ],
    )
    def sc_kernel(x_hbm, e_hbm, out_hbm, idx_v, rows_v, acc, isems, gsems):
        c = lax.axis_index("c")
        s = lax.axis_index("s")
        x_mine = x_hbm.at[c]
        n_chunks = jnp.where(c == 0, NCH0, NCH1)
        chunk_base = jnp.where(c == 0, s * NCH0, NS * NCH0 + s * NCH1)

        scope_zero = jax.named_scope("ph_zero"); scope_zero.__enter__()
        def zero_row(i, carry):
            for j in range(d_feat // 16):
                rows_v[0, i, pl.ds(j * 16, 16)] = jnp.zeros((16,), jnp.float32)
            return carry

        lax.fori_loop(0, K, zero_row, 0)
        row_base = s * rows_per_tile
        nfull = rows_per_tile // K
        for b in range(nfull):
            pltpu.sync_copy(rows_v.at[0], acc.at[pl.ds(row_base + b * K, K)])
        rem = rows_per_tile - nfull * K
        if rem:
            pltpu.sync_copy(
                rows_v.at[0, pl.ds(0, rem)],
                acc.at[pl.ds(row_base + nfull * K, rem)],
            )
        plsc.subcore_barrier()
        scope_zero.__exit__(None, None, None)
        scope_main = jax.named_scope("ph_main"); scope_main.__enter__()

        def idx_start(g, i):
            pltpu.async_copy(e_hbm.at[chunk_base + g], idx_v.at[i], isems[i])

        def idx_wait(i):
            pltpu.make_async_copy(e_hbm.at[0], idx_v.at[i], isems[i]).wait()

        def gather_start(g, i, b):
            pltpu.async_copy(x_mine.at[idx_v.at[i, 0]], rows_v.at[b], gsems[b])

        def gather_wait(b):
            pltpu.make_async_copy(
                x_mine.at[idx_v.at[0, 0]], rows_v.at[b], gsems[b]
            ).wait()

        def scatter(i, b):
            pltpu.sync_copy(rows_v.at[b], acc.at[idx_v.at[i, 1]], add=True)

        for g in range(NIDX):
            idx_start(g, g)
        for g in range(NBUF):
            idx_wait(g)
            gather_start(g, g, g)

        @pl.loop(0, n_chunks - NIDX, step=NIDX)
        def _(g0):
            for b in range(NIDX):
                g = g0 + b
                gather_wait(b % NBUF)
                scatter(b, b % NBUF)
                idx_start(g + NIDX, b)
                idx_wait((b + NBUF) % NIDX)
                gather_start(g + NBUF, (b + NBUF) % NIDX, b % NBUF)

        for b in range(NIDX):
            g = n_chunks - NIDX + b
            gather_wait(b % NBUF)
            scatter(b, b % NBUF)
            if b + NBUF < NIDX:
                idx_wait((b + NBUF) % NIDX)
                gather_start(g + NBUF, (b + NBUF) % NIDX, b % NBUF)
        plsc.subcore_barrier()
        scope_main.__exit__(None, None, None)

        with jax.named_scope("ph_writeback"):
            pltpu.sync_copy(
                acc.at[pl.ds(row_base, rows_per_tile)],
                out_hbm.at[c, pl.ds(row_base, rows_per_tile)],
            )

    return sc_kernel(x, edges)


def _finalize(x, acc):
    n_nodes, d_feat = x.shape
    br = 1000

    def body(x_ref, acc_ref, o_ref):
        o_ref[:, :d_feat] = x_ref[...]
        o_ref[:, d_feat:] = acc_ref[0] + acc_ref[1]

    return pl.pallas_call(
        body,
        grid=(n_nodes // br,),
        in_specs=[
            pl.BlockSpec((br, d_feat), lambda i: (i, 0)),
            pl.BlockSpec((NC, br, d_feat), lambda i: (0, i, 0)),
        ],
        out_specs=pl.BlockSpec((br, 2 * d_feat), lambda i: (i, 0)),
        out_shape=jax.ShapeDtypeStruct((n_nodes, 2 * d_feat), jnp.float32),
    )(x, acc)


@jax.jit
def _run(x, edges):
    xx = jnp.broadcast_to(x, (NC,) + x.shape)
    acc = _sc_scatter_add(xx, edges)
    return _finalize(x, acc)


def kernel(x, edge_index, batch_size):
    n_nodes = x.shape[0]
    n_edges = edge_index.shape[1]
    ei = edge_index.astype(jnp.int32)
    n_pad_rows = ((n_nodes + NS * 8 - 1) // (NS * 8)) * (NS * 8) - n_nodes
    n_padded = NS * (NCH0 + NCH1) * K
    pad = n_padded - n_edges
    src = jnp.concatenate([ei[0], jnp.zeros((pad,), jnp.int32)])
    dst = jnp.concatenate(
        [ei[1], n_nodes + (jnp.arange(pad, dtype=jnp.int32) % n_pad_rows)]
    )
    edges = jnp.stack(
        [src.reshape(-1, K), dst.reshape(-1, K)], axis=1
    )
    return _run(x, edges)
